# baseline (speedup 1.0000x reference)
import functools

import jax
import jax.numpy as jnp
from jax import lax
from jax.experimental import pallas as pl
from jax.experimental.pallas import tpu as pltpu
from jax.experimental.pallas import tpu_sc as plsc

VOCAB = 100000
D = 128
B = 1024
L = 200

NC = 2
NS = 16
LANES = 16
NW = NC * NS
N = B * L
TOK_PER_W = N // NW
KT = 128
CH = TOK_PER_W // KT
RROWS = 8
NB = 3


def _sc_body(x_hbm, tab_hbm, out_hbm,
             idg0, idg1, idg2, acc0, acc1, acc2,
             sr0, sr1, sr2, sg0, sg1, sg2, so0, so1, so2):
    cid = lax.axis_index("c")
    sid = lax.axis_index("s")
    wid = sid * NC + cid

    idg = (idg0, idg1, idg2)
    acc = (acc0, acc1, acc2)
    sr = (sr0, sr1, sr2)
    sg = (sg0, sg1, sg2)
    so = (so0, so1, so2)

    def idx_slice(t):
        r0 = pl.multiple_of(wid * (CH * RROWS) + t * RROWS, 8)
        return x_hbm.at[pl.ds(r0, RROWS)]

    def fire_idx(t, p):
        pltpu.async_copy(idx_slice(t), idg[p], sr[p])

    def wait_idx(t, p):
        pltpu.make_async_copy(idx_slice(t), idg[p], sr[p]).wait()

    def offset_add(p):
        gp = idg[p]
        for i in range(RROWS):
            for c in range(128 // LANES):
                sl = pl.ds(c * LANES, LANES)
                gp[i, sl] = gp[i, sl] + (i * VOCAB)

    def zero_acc(p):
        ap = acc[p]
        zv = jnp.zeros((LANES,), jnp.float32)

        def z_body(j, carry):
            for c in range(D // LANES):
                ap[j, pl.ds(c * LANES, LANES)] = zv
            return carry

        lax.fori_loop(0, KT, z_body, 0, unroll=4)

    def fire_gathers(p):
        for i in range(RROWS):
            pltpu.async_copy(tab_hbm.at[idg[p].at[i]], acc[p], sg[p],
                             add=True).wait()

    def wait_gathers(p):
        del p

    def out_slice(t):
        return out_hbm.at[pl.ds(pl.multiple_of(wid * TOK_PER_W + t * KT, KT), KT)]

    def fire_out(t, p):
        pltpu.async_copy(acc[p], out_slice(t), so[p])

    def wait_out(t, p):
        pltpu.make_async_copy(acc[p], out_slice(t), so[p]).wait()

    def prep(t2, p2, first):
        wait_idx(t2, p2)
        offset_add(p2)
        if not first:
            wait_out(t2 - NB, p2)
        zero_acc(p2)
        fire_gathers(p2)

    def half_iter(t, p, *, first=False, fire_idx_t5=True, prep_t2=True):
        wait_gathers(p)
        fire_out(t, p)
        if prep_t2:
            p2 = (p + 2) % NB
            prep(t + 2, p2, first)
            if fire_idx_t5:
                fire_idx(t + 5, p2)

    for t in range(NB):
        fire_idx(t, t)
    prep(0, 0, True)
    fire_idx(3, 0)
    prep(1, 1, True)
    fire_idx(4, 1)

    half_iter(0, 0, first=True)
    half_iter(1, 1)
    half_iter(2, 2)

    def steady(v, carry):
        t0 = 3 * v
        for r in range(NB):
            half_iter(t0 + r, r)
        return carry

    lax.fori_loop(1, 15, steady, 0)

    half_iter(45, 0, fire_idx_t5=False)
    half_iter(46, 1, fire_idx_t5=False)
    half_iter(47, 2, fire_idx_t5=False)
    half_iter(48, 0, prep_t2=False)
    half_iter(49, 1, prep_t2=False)
    wait_out(47, 2)
    wait_out(48, 0)
    wait_out(49, 1)


@jax.jit
def _sc_lookup_sum(xg, tab2d):
    mesh = plsc.VectorSubcoreMesh(core_axis_name="c", subcore_axis_name="s")
    f = functools.partial(
        pl.kernel,
        mesh=mesh,
        out_type=jax.ShapeDtypeStruct((N, D), jnp.float32),
        scratch_types=(
            [pltpu.VMEM((RROWS, 128), jnp.int32) for _ in range(NB)]
            + [pltpu.VMEM((KT, D), jnp.float32) for _ in range(NB)]
            + [pltpu.SemaphoreType.DMA for _ in range(3 * NB)]
        ),
    )(_sc_body)
    return f(xg, tab2d)


def kernel(x, tables):
    xg = (
        x.reshape(NW, CH, KT, 8)
        .transpose(0, 1, 3, 2)
        .reshape(NW * CH * RROWS, 128)
    )
    tab2d = tables.reshape(8 * VOCAB, D)
    out = _sc_lookup_sum(xg, tab2d)
    return out.reshape(B, L, D)


# speedup vs baseline: 1.4032x; 1.4032x over previous
import functools

import jax
import jax.numpy as jnp
from jax import lax
from jax.experimental import pallas as pl
from jax.experimental.pallas import tpu as pltpu
from jax.experimental.pallas import tpu_sc as plsc

VOCAB = 100000
D = 128
B = 1024
L = 200

NC = 2
NS = 16
LANES = 16
NW = NC * NS
N = B * L
TOK_PER_W = N // NW
KT = 128
CH = TOK_PER_W // KT
NP = CH // 2
RROWS = 8
NB = 4


def _sc_body(x_hbm, tab_hbm, out_hbm,
             idg0, idg1, idg2, idg3, acc0, acc1, acc2, acc3,
             sr0, sr1, sr2, sr3, sg0, sg1, sg2, sg3, so0, so1, so2, so3):
    cid = lax.axis_index("c")
    sid = lax.axis_index("s")
    wid = sid * NC + cid

    idg = (idg0, idg1, idg2, idg3)
    acc = (acc0, acc1, acc2, acc3)
    sr = (sr0, sr1, sr2, sr3)
    sg = (sg0, sg1, sg2, sg3)
    so = (so0, so1, so2, so3)

    def idx_slice(t):
        r0 = pl.multiple_of(wid * (CH * RROWS) + t * RROWS, 8)
        return x_hbm.at[pl.ds(r0, RROWS)]

    def fire_idx(t, p):
        pltpu.async_copy(idx_slice(t), idg[p], sr[p])

    def wait_idx(t, p):
        pltpu.make_async_copy(idx_slice(t), idg[p], sr[p]).wait()

    def offset_add(p):
        gp = idg[p]
        for i in range(RROWS):
            for c in range(128 // LANES):
                sl = pl.ds(c * LANES, LANES)
                gp[i, sl] = gp[i, sl] + (i * VOCAB)

    def zero_acc(p):
        ap = acc[p]
        zv = jnp.zeros((LANES,), jnp.float32)

        def z_body(j, carry):
            for c in range(D // LANES):
                ap[j, pl.ds(c * LANES, LANES)] = zv
            return carry

        lax.fori_loop(0, KT, z_body, 0, unroll=4)

    def fire_g(i, p):
        pltpu.async_copy(tab_hbm.at[idg[p].at[i]], acc[p], sg[p], add=True)

    def wait_g(i, p):
        pltpu.make_async_copy(tab_hbm.at[idg[p].at[i]], acc[p], sg[p]).wait()

    def out_slice(t):
        return out_hbm.at[pl.ds(pl.multiple_of(wid * TOK_PER_W + t * KT, KT), KT)]

    def fire_out(t, p):
        pltpu.async_copy(acc[p], out_slice(t), so[p])

    def wait_out(t, p):
        pltpu.make_async_copy(acc[p], out_slice(t), so[p]).wait()

    def pair_iter(u, pa, *, first=False, prep=True, stage=True):
        a = 2 * u
        b = a + 1
        pb = pa + 1
        pa2 = (pa + 2) % NB
        pb2 = pa2 + 1
        fire_g(0, pa)
        fire_g(0, pb)
        if prep:
            wait_idx(a + 2, pa2)
            offset_add(pa2)
            wait_idx(b + 2, pb2)
            offset_add(pb2)
            if not first:
                wait_out(a - 2, pa2)
                wait_out(b - 2, pb2)
            zero_acc(pa2)
            zero_acc(pb2)
        for i in range(RROWS - 1):
            wait_g(i, pa)
            fire_g(i + 1, pa)
            wait_g(i, pb)
            fire_g(i + 1, pb)
        wait_g(RROWS - 1, pa)
        fire_out(a, pa)
        wait_g(RROWS - 1, pb)
        fire_out(b, pb)
        if stage:
            fire_idx(a + 4, pa)
            fire_idx(b + 4, pb)

    for t in range(NB):
        fire_idx(t, t)
    wait_idx(0, 0)
    offset_add(0)
    zero_acc(0)
    wait_idx(1, 1)
    offset_add(1)
    zero_acc(1)

    pair_iter(0, 0, first=True)

    def steady(v, carry):
        u = 2 * v + 1
        pair_iter(u, 2)
        pair_iter(u + 1, 0)
        return carry

    lax.fori_loop(0, 11, steady, 0)

    pair_iter(23, 2, stage=False)
    pair_iter(24, 0, prep=False, stage=False)
    wait_out(46, 2)
    wait_out(47, 3)
    wait_out(48, 0)
    wait_out(49, 1)


@jax.jit
def _sc_lookup_sum(xg, tab2d):
    mesh = plsc.VectorSubcoreMesh(core_axis_name="c", subcore_axis_name="s")
    f = functools.partial(
        pl.kernel,
        mesh=mesh,
        out_type=jax.ShapeDtypeStruct((N, D), jnp.float32),
        scratch_types=(
            [pltpu.VMEM((RROWS, 128), jnp.int32) for _ in range(NB)]
            + [pltpu.VMEM((KT, D), jnp.float32) for _ in range(NB)]
            + [pltpu.SemaphoreType.DMA for _ in range(3 * NB)]
        ),
    )(_sc_body)
    return f(xg, tab2d)


def kernel(x, tables):
    xg = (
        x.reshape(NW, CH, KT, 8)
        .transpose(0, 1, 3, 2)
        .reshape(NW * CH * RROWS, 128)
    )
    tab2d = tables.reshape(8 * VOCAB, D)
    out = _sc_lookup_sum(xg, tab2d)
    return out.reshape(B, L, D)


# speedup vs baseline: 1.4434x; 1.0287x over previous
import functools

import jax
import jax.numpy as jnp
from jax import lax
from jax.experimental import pallas as pl
from jax.experimental.pallas import tpu as pltpu
from jax.experimental.pallas import tpu_sc as plsc

VOCAB = 100000
D = 128
B = 1024
L = 200

NC = 2
NS = 16
LANES = 16
NW = NC * NS
N = B * L
TOK_PER_W = N // NW
KT = 128
CH = TOK_PER_W // KT
NP = CH // 2
RROWS = 8
NB = 4


def _sc_body(x_hbm, tab_hbm, out_hbm,
             idg0, idg1, idg2, idg3, acc0, acc1, acc2, acc3,
             sr0, sr1, sr2, sr3, sg0, sg1, sg2, sg3, so0, so1, so2, so3):
    cid = lax.axis_index("c")
    sid = lax.axis_index("s")
    wid = sid * NC + cid

    idg = (idg0, idg1, idg2, idg3)
    acc = (acc0, acc1, acc2, acc3)
    sr = (sr0, sr1, sr2, sr3)
    sg = (sg0, sg1, sg2, sg3)
    so = (so0, so1, so2, so3)

    def idx_slice(t):
        r0 = pl.multiple_of(wid * (CH * RROWS) + t * RROWS, 8)
        return x_hbm.at[pl.ds(r0, RROWS)]

    def fire_idx(t, p):
        pltpu.async_copy(idx_slice(t), idg[p], sr[p])

    def wait_idx(t, p):
        pltpu.make_async_copy(idx_slice(t), idg[p], sr[p]).wait()

    def offset_add(p):
        gp = idg[p]
        for i in range(RROWS):
            for c in range(128 // LANES):
                sl = pl.ds(c * LANES, LANES)
                gp[i, sl] = gp[i, sl] + (i * VOCAB)

    def fire_g(i, p):
        pltpu.async_copy(tab_hbm.at[idg[p].at[i]], acc[p], sg[p], add=i > 0)

    def wait_g(i, p):
        pltpu.make_async_copy(tab_hbm.at[idg[p].at[i]], acc[p], sg[p]).wait()

    def out_slice(t):
        return out_hbm.at[pl.ds(pl.multiple_of(wid * TOK_PER_W + t * KT, KT), KT)]

    def fire_out(t, p):
        pltpu.async_copy(acc[p], out_slice(t), so[p])

    def wait_out(t, p):
        pltpu.make_async_copy(acc[p], out_slice(t), so[p]).wait()

    def pair_iter(u, pa, *, first=False, prep=True, stage=True):
        a = 2 * u
        b = a + 1
        pb = pa + 1
        pa2 = (pa + 2) % NB
        pb2 = pa2 + 1
        fire_g(0, pa)
        fire_g(0, pb)
        if prep:
            wait_idx(a + 2, pa2)
            offset_add(pa2)
            wait_idx(b + 2, pb2)
            offset_add(pb2)
            if not first:
                wait_out(a - 2, pa2)
                wait_out(b - 2, pb2)
        for i in range(RROWS - 1):
            wait_g(i, pa)
            fire_g(i + 1, pa)
            wait_g(i, pb)
            fire_g(i + 1, pb)
        wait_g(RROWS - 1, pa)
        fire_out(a, pa)
        wait_g(RROWS - 1, pb)
        fire_out(b, pb)
        if stage:
            fire_idx(a + 4, pa)
            fire_idx(b + 4, pb)

    for t in range(NB):
        fire_idx(t, t)
    wait_idx(0, 0)
    offset_add(0)
    wait_idx(1, 1)
    offset_add(1)

    pair_iter(0, 0, first=True)

    def steady(v, carry):
        u = 2 * v + 1
        pair_iter(u, 2)
        pair_iter(u + 1, 0)
        return carry

    lax.fori_loop(0, 11, steady, 0)

    pair_iter(23, 2, stage=False)
    pair_iter(24, 0, prep=False, stage=False)
    wait_out(46, 2)
    wait_out(47, 3)
    wait_out(48, 0)
    wait_out(49, 1)


@jax.jit
def _sc_lookup_sum(xg, tab2d):
    mesh = plsc.VectorSubcoreMesh(core_axis_name="c", subcore_axis_name="s")
    f = functools.partial(
        pl.kernel,
        mesh=mesh,
        out_type=jax.ShapeDtypeStruct((N, D), jnp.float32),
        scratch_types=(
            [pltpu.VMEM((RROWS, 128), jnp.int32) for _ in range(NB)]
            + [pltpu.VMEM((KT, D), jnp.float32) for _ in range(NB)]
            + [pltpu.SemaphoreType.DMA for _ in range(3 * NB)]
        ),
    )(_sc_body)
    return f(xg, tab2d)


def kernel(x, tables):
    xg = (
        x.reshape(NW, CH, KT, 8)
        .transpose(0, 1, 3, 2)
        .reshape(NW * CH * RROWS, 128)
    )
    tab2d = tables.reshape(8 * VOCAB, D)
    out = _sc_lookup_sum(xg, tab2d)
    return out.reshape(B, L, D)
